# Initial kernel scaffold; baseline (speedup 1.0000x reference)
import functools

import jax
import jax.numpy as jnp
from jax.experimental import pallas as pl
from jax.experimental.pallas import tpu as pltpu

DIM = 768
NUM_HEADS = 12
DH = DIM // NUM_HEADS
TOTAL_TOKENS = 320
NUM_EXPERTS = 8
TOPK = 2
N = 576
N3 = N // 3
KC = N3
KMAX = 13
BIG = 1e30
HI = jax.lax.Precision.HIGHEST


def _rank_ascending(s):
    n = s.shape[0]
    si = s[:, None]
    sj = s[None, :]
    ii = jax.lax.broadcasted_iota(jnp.float32, (n, n), 0)
    jj = jax.lax.broadcasted_iota(jnp.float32, (n, n), 1)
    lt = (sj < si).astype(jnp.float32)
    eq = ((sj == si) & (jj < ii)).astype(jnp.float32)
    return jnp.sum(lt + eq, axis=1)


def _rank_descending(s):
    n = s.shape[0]
    si = s[:, None]
    sj = s[None, :]
    ii = jax.lax.broadcasted_iota(jnp.float32, (n, n), 0)
    jj = jax.lax.broadcasted_iota(jnp.float32, (n, n), 1)
    gt = (sj > si).astype(jnp.float32)
    eq = ((sj == si) & (jj < ii)).astype(jnp.float32)
    return jnp.sum(gt + eq, axis=1)


def _onehot_from_rank(rank, n):
    pp = jax.lax.broadcasted_iota(jnp.float32, (n, n), 0)
    return (pp == rank[None, :]).astype(jnp.float32)


def _cluster_merge(xr, tw, noise, nf, kkf):
    m = xr.shape[0]
    sq = jnp.sum(xr * xr, axis=-1)
    d2 = sq[:, None] + sq[None, :] - 2.0 * jax.lax.dot_general(
        xr, xr, (((1,), (1,)), ((), ())), precision=HI)
    dist = jnp.sqrt(jnp.clip(d2, 0.0, None)) * (DIM ** -0.5)

    jj = jax.lax.broadcasted_iota(jnp.float32, (m, m), 1)
    dw = dist
    acc = jnp.zeros((m,), jnp.float32)
    for t in range(KMAX):
        mn = jnp.min(dw, axis=1)
        acc = acc + jnp.where(jnp.float32(t) < kkf, mn * mn, 0.0)
        eq = dw == mn[:, None]
        jmin = jnp.min(jnp.where(eq, jj, BIG), axis=1)
        dw = jnp.where(jj == jmin[:, None], BIG, dw)

    density = jnp.exp(-acc / kkf) + noise * 1e-6
    higher = (density[None, :] > density[:, None]).astype(jnp.float32)
    dist_max = jnp.max(dist)
    dmin = jnp.min(dist * higher + dist_max * (1.0 - higher), axis=1)
    score_c = dmin * density

    rank_d = _rank_descending(score_c)
    p2 = _onehot_from_rank(rank_d, m)
    dist_perm = jax.lax.dot_general(p2, dist, (((1,), (0,)), ((), ())),
                                    precision=HI)
    pp_col = jax.lax.broadcasted_iota(jnp.float32, (m, m), 0)
    dsel = dist_perm + jnp.where(pp_col < nf, 0.0, BIG)
    minv = jnp.min(dsel, axis=0)
    idxp = jnp.min(jnp.where(dsel == minv[None, :], pp_col, BIG), axis=0)
    idxf = jnp.where(rank_d < nf, rank_d, idxp)

    cc = jax.lax.broadcasted_iota(jnp.float32, (m, m), 0)
    conehot = (cc == idxf[None, :]).astype(jnp.float32)
    allw = jax.lax.dot_general(conehot, tw, (((1,), (0,)), ((), ())),
                               precision=HI) + 1e-6
    denom = jax.lax.dot_general(conehot, allw, (((0,), (0,)), ((), ())),
                                precision=HI)
    nw = tw / denom
    agg = jax.lax.dot_general(conehot, xr * nw, (((1,), (0,)), ((), ())),
                              precision=HI)
    valid = (jax.lax.broadcasted_iota(jnp.float32, (m, 1), 0)[:, 0] < nf)
    return agg, valid.astype(jnp.float32)


def _prep_kernel(x_ref, wq_ref, wsc_ref, bsc_ref, ws_ref, bs_ref,
                 wr_ref, br_ref, noise_ref,
                 agg_ref, valid_ref, q_ref, wsel_ref, topk_ref):
    x = x_ref[0]

    score = jnp.exp(
        jax.lax.dot_general(x, wsc_ref[...], (((1,), (0,)), ((), ())),
                            precision=HI)[:, 0] + bsc_ref[0, 0])
    rank = _rank_ascending(score)
    perm = _onehot_from_rank(rank, N)
    xs = jax.lax.dot_general(perm, x, (((1,), (0,)), ((), ())), precision=HI)
    ss = jax.lax.dot_general(perm, score[:, None], (((1,), (0,)), ((), ())),
                             precision=HI)

    ws = ws_ref[...]
    s_logits = jnp.sum(ss.reshape(3, N3) * ws.T, axis=1) + bs_ref[0, :]
    mx = jnp.max(s_logits)
    ex = jnp.exp(s_logits - mx)
    norm = ex / jnp.sum(ex)
    agg_num = jnp.clip(TOTAL_TOKENS * norm, 16.0, float(TOTAL_TOKENS))
    nf3 = jnp.floor(agg_num)
    kkf3 = jnp.maximum(1.0, jnp.floor(jnp.sqrt(nf3)))

    q_ref[0] = jax.lax.dot_general(x, wq_ref[...], (((1,), (0,)), ((), ())),
                                   precision=HI)

    xm = jnp.mean(x, axis=0)[None, :]
    wl = jax.lax.dot_general(xm, wr_ref[...], (((1,), (0,)), ((), ())),
                             precision=HI) + br_ref[...]
    wts = jax.nn.sigmoid(wl)
    ie = jax.lax.broadcasted_iota(jnp.float32, (1, NUM_EXPERTS), 1)
    m1 = jnp.max(wts)
    i1 = jnp.min(jnp.where(wts == m1, ie, BIG))
    wts2 = jnp.where(ie == i1, -BIG, wts)
    m2 = jnp.max(wts2)
    i2 = jnp.min(jnp.where(wts2 == m2, ie, BIG))
    wsel_ref[0, 0, 0] = m1
    wsel_ref[0, 0, 1] = m2
    topk_ref[0, 0, 0] = i1.astype(jnp.int32)
    topk_ref[0, 0, 1] = i2.astype(jnp.int32)

    for r in range(3):
        xr = xs[r * N3:(r + 1) * N3, :]
        tw = ss[r * N3:(r + 1) * N3, :]
        agg, valid = _cluster_merge(xr, tw, noise_ref[0, r, 0, :],
                                    nf3[r], kkf3[r])
        agg_ref[0, r * N3:(r + 1) * N3, :] = agg
        valid_ref[0, r, 0, :] = valid


def _attn_kernel(topk_ref, wsel_ref, agg_ref, q_ref, valid_ref,
                 wkv_ref, wproj_ref, bproj_ref, out_ref):
    i = pl.program_id(0)
    j = pl.program_id(1)
    agg = agg_ref[0]
    q = q_ref[0]
    valid = valid_ref[0, 0, :][None, :]
    w = wsel_ref[i, j]

    kv = jax.lax.dot_general(agg, wkv_ref[0], (((1,), (0,)), ((), ())),
                             precision=HI)
    sc = DH ** -0.5
    parts = []
    for h in range(NUM_HEADS):
        qh = q[:, h * DH:(h + 1) * DH]
        kh = kv[:, h * DH:(h + 1) * DH]
        vh = kv[:, DIM + h * DH:DIM + (h + 1) * DH]
        logits = jax.lax.dot_general(qh, kh, (((1,), (1,)), ((), ())),
                                     precision=HI) * sc
        logits = jnp.where(valid > 0.0, logits, -BIG)
        mx = jnp.max(logits, axis=1, keepdims=True)
        p = jnp.exp(logits - mx)
        attn = p / jnp.sum(p, axis=1, keepdims=True)
        parts.append(jax.lax.dot_general(attn, vh, (((1,), (0,)), ((), ())),
                                         precision=HI))
    o = jnp.concatenate(parts, axis=1) * w

    @pl.when(j == 0)
    def _():
        out_ref[0] = o

    @pl.when(j == 1)
    def _():
        acc = out_ref[0] + o
        out_ref[0] = jax.lax.dot_general(
            acc, wproj_ref[...], (((1,), (0,)), ((), ())),
            precision=HI) + bproj_ref[...]


@jax.jit
def kernel(x, W_q, W_kv, W_route, b_route, W_proj, b_proj, W_score, b_score,
           W_s1, b_s1, W_s2, b_s2, W_s3, b_s3):
    B = x.shape[0]

    rkey = jax.random.key(42)
    noise = jnp.stack([
        jax.random.uniform(jax.random.fold_in(rkey, t), (KC,), jnp.float32)
        for t in range(B * 3)]).reshape(B, 3, 1, KC)

    ws = jnp.concatenate([W_s1, W_s2, W_s3], axis=1)
    bs = jnp.stack([b_s1[0], b_s2[0], b_s3[0]])[None, :]

    agg, valid, q, wsel, topk = pl.pallas_call(
        _prep_kernel,
        grid=(B,),
        in_specs=[
            pl.BlockSpec((1, N, DIM), lambda i: (i, 0, 0)),
            pl.BlockSpec((DIM, DIM), lambda i: (0, 0)),
            pl.BlockSpec((DIM, 1), lambda i: (0, 0)),
            pl.BlockSpec((1, 1), lambda i: (0, 0)),
            pl.BlockSpec((N3, 3), lambda i: (0, 0)),
            pl.BlockSpec((1, 3), lambda i: (0, 0)),
            pl.BlockSpec((DIM, NUM_EXPERTS), lambda i: (0, 0)),
            pl.BlockSpec((1, NUM_EXPERTS), lambda i: (0, 0)),
            pl.BlockSpec((1, 3, 1, KC), lambda i: (i, 0, 0, 0)),
        ],
        out_specs=[
            pl.BlockSpec((1, N, DIM), lambda i: (i, 0, 0)),
            pl.BlockSpec((1, 3, 1, KC), lambda i: (i, 0, 0, 0)),
            pl.BlockSpec((1, N, DIM), lambda i: (i, 0, 0)),
            pl.BlockSpec((1, 1, TOPK), lambda i: (i, 0, 0)),
            pl.BlockSpec((1, 1, TOPK), lambda i: (i, 0, 0)),
        ],
        out_shape=[
            jax.ShapeDtypeStruct((B, N, DIM), jnp.float32),
            jax.ShapeDtypeStruct((B, 3, 1, KC), jnp.float32),
            jax.ShapeDtypeStruct((B, N, DIM), jnp.float32),
            jax.ShapeDtypeStruct((B, 1, TOPK), jnp.float32),
            jax.ShapeDtypeStruct((B, 1, TOPK), jnp.int32),
        ],
    )(x, W_q, W_score, b_score.reshape(1, 1), ws, bs,
      W_route, b_route.reshape(1, NUM_EXPERTS), noise)

    topk_flat = topk.reshape(B, TOPK)
    wsel_flat = wsel.reshape(B, TOPK)
    valid_flat = valid.reshape(B, 1, N)

    res = pl.pallas_call(
        _attn_kernel,
        grid_spec=pltpu.PrefetchScalarGridSpec(
            num_scalar_prefetch=2,
            grid=(B, TOPK),
            in_specs=[
                pl.BlockSpec((1, N, DIM), lambda i, j, s0, s1: (i, 0, 0)),
                pl.BlockSpec((1, N, DIM), lambda i, j, s0, s1: (i, 0, 0)),
                pl.BlockSpec((1, 1, N), lambda i, j, s0, s1: (i, 0, 0)),
                pl.BlockSpec((1, DIM, 2 * DIM),
                             lambda i, j, s0, s1: (s0[i, j], 0, 0)),
                pl.BlockSpec((DIM, DIM), lambda i, j, s0, s1: (0, 0)),
                pl.BlockSpec((1, DIM), lambda i, j, s0, s1: (0, 0)),
            ],
            out_specs=pl.BlockSpec((1, N, DIM), lambda i, j, s0, s1: (i, 0, 0)),
        ),
        out_shape=jax.ShapeDtypeStruct((B, N, DIM), jnp.float32),
    )(topk_flat, wsel_flat, agg, q, valid_flat,
      W_kv, W_proj, b_proj.reshape(1, DIM))

    return res


# speedup vs baseline: 5.5878x; 5.5878x over previous
import jax
import jax.numpy as jnp
from jax.experimental import pallas as pl
from jax.experimental.pallas import tpu as pltpu

DIM = 768
NUM_HEADS = 12
DH = DIM // NUM_HEADS
TOTAL_TOKENS = 320
NUM_EXPERTS = 8
TOPK = 2
N = 576
N3 = N // 3
KC = N3
KMAX = 13
BIG = 1e30
HI = jax.lax.Precision.HIGHEST


def _iota(shape, dim):
    return jax.lax.broadcasted_iota(jnp.int32, shape, dim).astype(jnp.float32)


def _eye(n):
    return (jax.lax.broadcasted_iota(jnp.int32, (n, n), 0)
            == jax.lax.broadcasted_iota(jnp.int32, (n, n), 1)).astype(
                jnp.float32)


def _to_row(col, eye):
    return jax.lax.dot_general(col, eye, (((0,), (0,)), ((), ())),
                               precision=HI)


def _mm(a, b, prec=None):
    return jax.lax.dot_general(a, b, (((1,), (0,)), ((), ())), precision=prec)


def _mm_tl(a, b, prec=HI):
    return jax.lax.dot_general(a, b, (((0,), (0,)), ((), ())), precision=prec)


def _prep_kernel(x_ref, wq_ref, wsc_ref, bsc_ref, ws_ref, bs_ref,
                 wr_ref, br_ref,
                 xs_ref, ss_ref, nk_ref, q_ref, wsel_ref, topk_ref):
    x = x_ref[0]
    eye = _eye(N)

    s_col = jnp.exp(_mm(x, wsc_ref[...]) + bsc_ref[...])
    s_row = _to_row(s_col, eye)
    ii = _iota((N, N), 0)
    jj = _iota((N, N), 1)
    lt = (s_row < s_col).astype(jnp.float32)
    eq = ((s_row == s_col) & (jj < ii)).astype(jnp.float32)
    rank_col = jnp.sum(lt + eq, axis=1, keepdims=True)
    q_onehot = (rank_col == jj).astype(jnp.float32)
    xs = _mm_tl(q_onehot, x)
    ss = _mm_tl(q_onehot, s_col)
    xs_ref[0] = xs
    ss_ref[0] = ss

    s_logits = _mm_tl(ss, ws_ref[...], prec=None) + bs_ref[...]
    mx = jnp.max(s_logits, axis=1, keepdims=True)
    ex = jnp.exp(s_logits - mx)
    norm = ex / jnp.sum(ex, axis=1, keepdims=True)
    agg_num = jnp.clip(TOTAL_TOKENS * norm, 16.0, float(TOTAL_TOKENS))
    nf3 = jnp.floor(agg_num)
    kkf3 = jnp.maximum(1.0, jnp.floor(jnp.sqrt(nf3)))
    nk_ref[0] = jnp.concatenate(
        [nf3, kkf3, jnp.zeros((1, 2), jnp.float32)], axis=1)

    q_ref[0] = _mm(x, wq_ref[...])

    xm = jnp.mean(x, axis=0, keepdims=True)
    wts = jax.nn.sigmoid(_mm(xm, wr_ref[...]) + br_ref[...])
    ie = _iota((1, NUM_EXPERTS), 1)
    m1 = jnp.max(wts, axis=1, keepdims=True)
    i1 = jnp.min(jnp.where(wts == m1, ie, BIG), axis=1, keepdims=True)
    wts2 = jnp.where(ie == i1, -BIG, wts)
    m2 = jnp.max(wts2, axis=1, keepdims=True)
    i2 = jnp.min(jnp.where(wts2 == m2, ie, BIG), axis=1, keepdims=True)
    wsel_ref[0] = jnp.concatenate([m1, m2], axis=1)
    topk_ref[0] = jnp.concatenate([i1, i2], axis=1).astype(jnp.int32)


def _cluster_kernel(nk_ref, xs_ref, ss_ref, noise_ref, agg_ref, valid_ref):
    i = pl.program_id(0)
    r = pl.program_id(1)
    xr = xs_ref[0]
    tw = ss_ref[0]
    noise_col = noise_ref[0, 0]
    nf = nk_ref[i, r]
    kkf = nk_ref[i, r + 3]
    m = KC
    eye = _eye(m)

    sq_col = jnp.sum(xr * xr, axis=1, keepdims=True)
    sq_row = _to_row(sq_col, eye)
    d2 = sq_col + sq_row - 2.0 * jax.lax.dot_general(
        xr, xr, (((1,), (1,)), ((), ())))
    dist = jnp.sqrt(jnp.clip(d2, 0.0, None)) * (DIM ** -0.5)

    ii = _iota((m, m), 0)
    jj = _iota((m, m), 1)

    def knn_body(t, carry):
        dw, acc = carry
        mn = jnp.min(dw, axis=1, keepdims=True)
        acc = acc + jnp.where(t.astype(jnp.float32) < kkf, mn * mn, 0.0)
        eqm = dw == mn
        jmin = jnp.min(jnp.where(eqm, jj, BIG), axis=1, keepdims=True)
        dw = jnp.where(jj == jmin, BIG, dw)
        return dw, acc

    _, acc = jax.lax.fori_loop(0, KMAX, knn_body,
                               (dist, jnp.zeros((m, 1), jnp.float32)))

    den_col = jnp.exp(-acc / kkf) + noise_col * 1e-6
    den_row = _to_row(den_col, eye)
    higher = (den_row > den_col).astype(jnp.float32)
    dist_max = jnp.max(dist)
    dmin = jnp.min(dist * higher + dist_max * (1.0 - higher), axis=1,
                   keepdims=True)
    sc_col = dmin * den_col

    sc_row = _to_row(sc_col, eye)
    gt = (sc_row > sc_col).astype(jnp.float32)
    eqc = ((sc_row == sc_col) & (jj < ii)).astype(jnp.float32)
    rank_col = jnp.sum(gt + eqc, axis=1, keepdims=True)
    q_onehot = (rank_col == jj).astype(jnp.float32)
    dist_perm = _mm_tl(q_onehot, dist)
    dsel = dist_perm + jnp.where(ii < nf, 0.0, BIG)
    minv = jnp.min(dsel, axis=0, keepdims=True)
    idxp = jnp.min(jnp.where(dsel == minv, ii, BIG), axis=0, keepdims=True)
    rank_row = _to_row(rank_col, eye)
    idxf = jnp.where(rank_row < nf, rank_row, idxp)

    conehot = (ii == idxf).astype(jnp.float32)
    allw = _mm(conehot, tw) + 1e-6
    denom = _mm_tl(conehot, allw)
    nw = tw / denom
    agg_ref[0] = _mm(conehot, xr * nw)
    valid_ref[0, 0, 0:1, :] = (_iota((1, m), 1) < nf).astype(jnp.float32)


def _attn_kernel(topk_ref, wsel_ref, agg_ref, q_ref, valid_ref,
                 wkv_ref, wproj_ref, bproj_ref, out_ref):
    i = pl.program_id(0)
    j = pl.program_id(1)
    agg = agg_ref[0]
    q = q_ref[0]
    valid = valid_ref[0]
    w = wsel_ref[i, j]

    kv = _mm(agg, wkv_ref[0])
    sc = DH ** -0.5
    parts = []
    for h in range(NUM_HEADS):
        qh = q[:, h * DH:(h + 1) * DH]
        kh = kv[:, h * DH:(h + 1) * DH]
        vh = kv[:, DIM + h * DH:DIM + (h + 1) * DH]
        logits = jax.lax.dot_general(qh, kh, (((1,), (1,)), ((), ()))) * sc
        logits = jnp.where(valid > 0.0, logits, -BIG)
        mx = jnp.max(logits, axis=1, keepdims=True)
        p = jnp.exp(logits - mx)
        attn = p / jnp.sum(p, axis=1, keepdims=True)
        parts.append(_mm(attn, vh))
    o = jnp.concatenate(parts, axis=1) * w

    @pl.when(j == 0)
    def _():
        out_ref[0] = o

    @pl.when(j == 1)
    def _():
        acc = out_ref[0] + o
        out_ref[0] = _mm(acc, wproj_ref[...]) + bproj_ref[...]


@jax.jit
def kernel(x, W_q, W_kv, W_route, b_route, W_proj, b_proj, W_score, b_score,
           W_s1, b_s1, W_s2, b_s2, W_s3, b_s3):
    B = x.shape[0]

    rkey = jax.random.key(42)
    noise = jnp.stack([
        jax.random.uniform(jax.random.fold_in(rkey, t), (KC,), jnp.float32)
        for t in range(B * 3)]).reshape(B, 3, KC, 1)

    zero = jnp.zeros((N3, 1), jnp.float32)
    wsfull = jnp.concatenate([
        jnp.concatenate([W_s1, zero, zero], axis=0),
        jnp.concatenate([zero, W_s2, zero], axis=0),
        jnp.concatenate([zero, zero, W_s3], axis=0)], axis=1)
    bs = jnp.stack([b_s1[0], b_s2[0], b_s3[0]])[None, :]

    xs, ss, nk, q, wsel, topk = pl.pallas_call(
        _prep_kernel,
        grid=(B,),
        in_specs=[
            pl.BlockSpec((1, N, DIM), lambda i: (i, 0, 0)),
            pl.BlockSpec((DIM, DIM), lambda i: (0, 0)),
            pl.BlockSpec((DIM, 1), lambda i: (0, 0)),
            pl.BlockSpec((1, 1), lambda i: (0, 0)),
            pl.BlockSpec((N, 3), lambda i: (0, 0)),
            pl.BlockSpec((1, 3), lambda i: (0, 0)),
            pl.BlockSpec((DIM, NUM_EXPERTS), lambda i: (0, 0)),
            pl.BlockSpec((1, NUM_EXPERTS), lambda i: (0, 0)),
        ],
        out_specs=[
            pl.BlockSpec((1, N, DIM), lambda i: (i, 0, 0)),
            pl.BlockSpec((1, N, 1), lambda i: (i, 0, 0)),
            pl.BlockSpec((1, 1, NUM_EXPERTS), lambda i: (i, 0, 0)),
            pl.BlockSpec((1, N, DIM), lambda i: (i, 0, 0)),
            pl.BlockSpec((1, 1, TOPK), lambda i: (i, 0, 0)),
            pl.BlockSpec((1, 1, TOPK), lambda i: (i, 0, 0)),
        ],
        out_shape=[
            jax.ShapeDtypeStruct((B, N, DIM), jnp.float32),
            jax.ShapeDtypeStruct((B, N, 1), jnp.float32),
            jax.ShapeDtypeStruct((B, 1, NUM_EXPERTS), jnp.float32),
            jax.ShapeDtypeStruct((B, N, DIM), jnp.float32),
            jax.ShapeDtypeStruct((B, 1, TOPK), jnp.float32),
            jax.ShapeDtypeStruct((B, 1, TOPK), jnp.int32),
        ],
    )(x, W_q, W_score, b_score.reshape(1, 1), wsfull, bs,
      W_route, b_route.reshape(1, NUM_EXPERTS))

    agg, valid = pl.pallas_call(
        _cluster_kernel,
        grid_spec=pltpu.PrefetchScalarGridSpec(
            num_scalar_prefetch=1,
            grid=(B, 3),
            in_specs=[
                pl.BlockSpec((1, N3, DIM), lambda i, r, s: (i, r, 0)),
                pl.BlockSpec((1, N3, 1), lambda i, r, s: (i, r, 0)),
                pl.BlockSpec((1, 1, KC, 1), lambda i, r, s: (i, r, 0, 0)),
            ],
            out_specs=[
                pl.BlockSpec((1, N3, DIM), lambda i, r, s: (i, r, 0)),
                pl.BlockSpec((1, 1, 1, KC), lambda i, r, s: (i, r, 0, 0)),
            ],
        ),
        out_shape=[
            jax.ShapeDtypeStruct((B, N, DIM), jnp.float32),
            jax.ShapeDtypeStruct((B, 3, 1, KC), jnp.float32),
        ],
    )(nk.reshape(B, NUM_EXPERTS), xs, ss, noise)

    topk_flat = topk.reshape(B, TOPK)
    wsel_flat = wsel.reshape(B, TOPK)
    valid_flat = valid.reshape(B, 1, N)

    res = pl.pallas_call(
        _attn_kernel,
        grid_spec=pltpu.PrefetchScalarGridSpec(
            num_scalar_prefetch=2,
            grid=(B, TOPK),
            in_specs=[
                pl.BlockSpec((1, N, DIM), lambda i, j, s0, s1: (i, 0, 0)),
                pl.BlockSpec((1, N, DIM), lambda i, j, s0, s1: (i, 0, 0)),
                pl.BlockSpec((1, 1, N), lambda i, j, s0, s1: (i, 0, 0)),
                pl.BlockSpec((1, DIM, 2 * DIM),
                             lambda i, j, s0, s1: (s0[i, j], 0, 0)),
                pl.BlockSpec((DIM, DIM), lambda i, j, s0, s1: (0, 0)),
                pl.BlockSpec((1, DIM), lambda i, j, s0, s1: (0, 0)),
            ],
            out_specs=pl.BlockSpec((1, N, DIM), lambda i, j, s0, s1: (i, 0, 0)),
        ),
        out_shape=jax.ShapeDtypeStruct((B, N, DIM), jnp.float32),
    )(topk_flat, wsel_flat, agg, q, valid_flat,
      W_kv, W_proj, b_proj.reshape(1, DIM))

    return res
